# Initial kernel scaffold; baseline (speedup 1.0000x reference)
import jax
import jax.numpy as jnp
from jax.experimental import pallas as pl


def kernel(data, emb, W1, b1, W2, b2):
    raise NotImplementedError("write your pallas kernel here")


# speedup vs baseline: 1.8072x; 1.8072x over previous
import functools

import jax
import jax.numpy as jnp
from jax import lax
from jax.experimental import pallas as pl
from jax.experimental.pallas import tpu as pltpu
from jax.experimental.pallas import tpu_sc as plsc

B, NA, L = 1024, 18, 200
D = 64
H1, H2 = 2048, 512
S = B * NA
NW = 32
SPW = S // NW
HALF = L // 2
UNROLL = 8


def _sc_mean(data_r, emb):
    mesh = plsc.VectorSubcoreMesh(core_axis_name="c", subcore_axis_name="s")

    @functools.partial(
        pl.kernel,
        out_type=jax.ShapeDtypeStruct((S, D), jnp.float32),
        mesh=mesh,
        scratch_types=[
            pltpu.VMEM((2, HALF), jnp.int32),
            pltpu.VMEM((2, HALF), jnp.int32),
            pltpu.VMEM((L, D), jnp.float32),
            pltpu.VMEM((L, D), jnp.float32),
            pltpu.VMEM((SPW, D), jnp.float32),
            pltpu.SemaphoreType.DMA,
            pltpu.SemaphoreType.DMA,
        ],
        compiler_params=pltpu.CompilerParams(use_tc_tiling_on_sc=False),
    )
    def k(data_hbm, emb_hbm, out_hbm, idx0, idx1, rows0, rows1, ostage,
          sem0, sem1):
        cid = lax.axis_index("c")
        sid = lax.axis_index("s")
        wid = sid * 2 + cid
        base = wid * SPW

        idx_bufs = (idx0, idx1)
        rows_bufs = (rows0, rows1)
        sems = (sem0, sem1)

        def stage_and_fire(l, b):
            pltpu.sync_copy(data_hbm.at[base + l], idx_bufs[b])
            for j in range(2):
                pltpu.async_copy(
                    emb_hbm.at[idx_bufs[b].at[j]],
                    rows_bufs[b].at[pl.ds(j * HALF, HALF)],
                    sems[b])

        def wait_gathers(b):
            for j in range(2):
                pltpu.make_async_copy(
                    emb_hbm.at[idx_bufs[b].at[j]],
                    rows_bufs[b].at[pl.ds(j * HALF, HALF)],
                    sems[b]).wait()

        def accumulate(l, b):
            rows = rows_bufs[b]

            def body(i, acc):
                accs = list(acc)
                for u in range(UNROLL):
                    r = i * UNROLL + u
                    for q in range(4):
                        accs[q] = accs[q] + rows[r, pl.ds(q * 16, 16)]
                return tuple(accs)

            z = jnp.zeros((16,), jnp.float32)
            a = lax.fori_loop(0, L // UNROLL, body, (z, z, z, z))
            for q in range(4):
                ostage[l, pl.ds(q * 16, 16)] = a[q] * (1.0 / L)

        stage_and_fire(0, 0)

        @pl.loop(0, SPW // 2)
        def _(p):
            s0 = p * 2
            stage_and_fire(s0 + 1, 1)
            wait_gathers(0)
            accumulate(s0, 0)

            @pl.when(p < SPW // 2 - 1)
            def _():
                stage_and_fire(s0 + 2, 0)

            wait_gathers(1)
            accumulate(s0 + 1, 1)

        pltpu.sync_copy(ostage, out_hbm.at[pl.ds(base, SPW)])

    return k(data_r, emb)


def _fold_weights(W1, b1r, W2, b2r):
    def body(w1_ref, b1_ref, w2_ref, b2_ref, w12_ref, bias_ref):
        w12_ref[...] = jnp.dot(w1_ref[...], w2_ref[...],
                               preferred_element_type=jnp.float32)
        bias_ref[...] = jnp.dot(b1_ref[...], w2_ref[...],
                                preferred_element_type=jnp.float32) + b2_ref[...]

    return pl.pallas_call(
        body,
        out_shape=(jax.ShapeDtypeStruct((D, H2), jnp.float32),
                   jax.ShapeDtypeStruct((1, H2), jnp.float32)),
    )(W1, b1r, W2, b2r)


def _mlp(x, w12, bias):
    blk = 1024

    def body(x_ref, w12_ref, bias_ref, o_ref):
        o_ref[...] = jnp.dot(x_ref[...], w12_ref[...],
                             preferred_element_type=jnp.float32) + bias_ref[...]

    return pl.pallas_call(
        body,
        grid=(S // blk,),
        in_specs=[
            pl.BlockSpec((blk, D), lambda i: (i, 0)),
            pl.BlockSpec((D, H2), lambda i: (0, 0)),
            pl.BlockSpec((1, H2), lambda i: (0, 0)),
        ],
        out_specs=pl.BlockSpec((blk, H2), lambda i: (i, 0)),
        out_shape=jax.ShapeDtypeStruct((S, H2), jnp.float32),
    )(x, w12, bias)


def kernel(data, emb, W1, b1, W2, b2):
    data_r = data.astype(jnp.int32).reshape(S, 2, HALF)
    x = _sc_mean(data_r, emb)
    w12, bias = _fold_weights(W1, b1.reshape(1, H1), W2, b2.reshape(1, H2))
    out = _mlp(x, w12, bias)
    return out.reshape(B, NA, H2)


# speedup vs baseline: 1.9869x; 1.0994x over previous
import functools

import jax
import jax.numpy as jnp
from jax import lax
from jax.experimental import pallas as pl
from jax.experimental.pallas import tpu as pltpu
from jax.experimental.pallas import tpu_sc as plsc

B, NA, L = 1024, 18, 200
D = 64
H1, H2 = 2048, 512
S = B * NA
NW = 32
SPW = S // NW
HALF = L // 2
UNROLL = 10
IDXB = 36
NCH = SPW // IDXB


def _sc_mean(data_r, emb):
    mesh = plsc.VectorSubcoreMesh(core_axis_name="c", subcore_axis_name="s")

    @functools.partial(
        pl.kernel,
        out_type=jax.ShapeDtypeStruct((S // 2, 2 * D), jnp.float32),
        mesh=mesh,
        scratch_types=[
            pltpu.VMEM((IDXB, 2, HALF), jnp.int32),
            pltpu.VMEM((IDXB, 2, HALF), jnp.int32),
            pltpu.VMEM((L, D), jnp.float32),
            pltpu.VMEM((L, D), jnp.float32),
            pltpu.VMEM((SPW // 2, 2 * D), jnp.float32),
            pltpu.SemaphoreType.DMA,
            pltpu.SemaphoreType.DMA,
            pltpu.SemaphoreType.DMA,
            pltpu.SemaphoreType.DMA,
        ],
        compiler_params=pltpu.CompilerParams(use_tc_tiling_on_sc=False),
    )
    def k(data_hbm, emb_hbm, out_hbm, idxc0, idxc1, rows0, rows1, ostage,
          gsem0, gsem1, isem0, isem1):
        cid = lax.axis_index("c")
        sid = lax.axis_index("s")
        wid = sid * 2 + cid
        base = wid * SPW

        idxc = (idxc0, idxc1)
        rows_bufs = (rows0, rows1)
        gsems = (gsem0, gsem1)
        isems = (isem0, isem1)

        def stage_chunk(c, par):
            pltpu.async_copy(
                data_hbm.at[pl.ds(base + c * IDXB, IDXB)], idxc[par],
                isems[par])

        def wait_chunk(c, par):
            pltpu.make_async_copy(
                data_hbm.at[pl.ds(base + c * IDXB, IDXB)], idxc[par],
                isems[par]).wait()

        def fire_seg(ls, b, par):
            for j in range(2):
                pltpu.async_copy(
                    emb_hbm.at[idxc[par].at[ls, j]],
                    rows_bufs[b].at[pl.ds(j * HALF, HALF)],
                    gsems[b])

        def wait_gathers(ls, b, par):
            for j in range(2):
                pltpu.make_async_copy(
                    emb_hbm.at[idxc[par].at[ls, j]],
                    rows_bufs[b].at[pl.ds(j * HALF, HALF)],
                    gsems[b]).wait()

        def accumulate(row, col, b):
            rows = rows_bufs[b]

            def body(i, acc):
                accs = list(acc)
                for u in range(UNROLL):
                    r = i * UNROLL + u
                    for q in range(4):
                        accs[q] = accs[q] + rows[r, pl.ds(q * 16, 16)]
                return tuple(accs)

            z = jnp.zeros((16,), jnp.float32)
            a = lax.fori_loop(0, L // UNROLL, body, (z, z, z, z))
            for q in range(4):
                ostage[row, pl.ds(col + q * 16, 16)] = a[q] * (1.0 / L)

        stage_chunk(0, 0)

        @pl.loop(0, NCH, step=2)
        def _(c2):
            for cp in range(2):
                c = c2 + cp
                cbase = c * IDXB

                @pl.when(c + 1 < NCH)
                def _():
                    stage_chunk(c + 1, 1 - cp)

                wait_chunk(c, cp)
                fire_seg(0, 0, cp)

                @pl.loop(0, IDXB // 2)
                def _(p):
                    s0 = p * 2
                    orow = cbase // 2 + p
                    fire_seg(s0 + 1, 1, cp)
                    wait_gathers(s0, 0, cp)
                    accumulate(orow, 0, 0)

                    @pl.when(p < IDXB // 2 - 1)
                    def _():
                        fire_seg(s0 + 2, 0, cp)

                    wait_gathers(s0 + 1, 1, cp)
                    accumulate(orow, D, 1)

        pltpu.sync_copy(ostage, out_hbm.at[pl.ds(wid * (SPW // 2), SPW // 2)])

    return k(data_r, emb)


def _fold_weights(W1, b1r, W2, b2r):
    def body(w1_ref, b1_ref, w2_ref, b2_ref, w12_ref, bias_ref):
        w12_ref[...] = jnp.dot(w1_ref[...], w2_ref[...],
                               preferred_element_type=jnp.float32)
        bias_ref[...] = jnp.dot(b1_ref[...], w2_ref[...],
                                preferred_element_type=jnp.float32) + b2_ref[...]

    return pl.pallas_call(
        body,
        out_shape=(jax.ShapeDtypeStruct((D, H2), jnp.float32),
                   jax.ShapeDtypeStruct((1, H2), jnp.float32)),
    )(W1, b1r, W2, b2r)


def _mlp(x, w12, bias):
    blk = 1024

    def body(x_ref, w12_ref, bias_ref, o_ref):
        o_ref[...] = jnp.dot(x_ref[...], w12_ref[...],
                             preferred_element_type=jnp.float32) + bias_ref[...]

    return pl.pallas_call(
        body,
        grid=(S // blk,),
        in_specs=[
            pl.BlockSpec((blk, D), lambda i: (i, 0)),
            pl.BlockSpec((D, H2), lambda i: (0, 0)),
            pl.BlockSpec((1, H2), lambda i: (0, 0)),
        ],
        out_specs=pl.BlockSpec((blk, H2), lambda i: (i, 0)),
        out_shape=jax.ShapeDtypeStruct((S, H2), jnp.float32),
    )(x, w12, bias)


def kernel(data, emb, W1, b1, W2, b2):
    data_r = data.astype(jnp.int32).reshape(S, 2, HALF)
    x = _sc_mean(data_r, emb).reshape(S, D)
    w12, bias = _fold_weights(W1, b1.reshape(1, H1), W2, b2.reshape(1, H2))
    out = _mlp(x, w12, bias)
    return out.reshape(B, NA, H2)


# speedup vs baseline: 2.9463x; 1.4828x over previous
import functools

import jax
import jax.numpy as jnp
from jax import lax
from jax.experimental import pallas as pl
from jax.experimental.pallas import tpu as pltpu
from jax.experimental.pallas import tpu_sc as plsc

B, NA, L = 1024, 18, 200
D = 64
H1, H2 = 2048, 512
VOCAB = 1000000
RP_CB = 8192
VPAD = 123 * RP_CB
S = B * NA
NW = 32
SPW = S // NW
SPLITS = (0, 104, 200)
UNROLL = 10
IDXB = 36
NCH = SPW // IDXB


def _sc_mean(data_r, emb):
    mesh = plsc.VectorSubcoreMesh(core_axis_name="c", subcore_axis_name="s")

    @functools.partial(
        pl.kernel,
        out_type=jax.ShapeDtypeStruct((S // 2, 2 * D), jnp.float32),
        mesh=mesh,
        scratch_types=[
            pltpu.VMEM((IDXB, L), jnp.int32),
            pltpu.VMEM((IDXB, L), jnp.int32),
            pltpu.VMEM((L, D), jnp.float32),
            pltpu.VMEM((L, D), jnp.float32),
            pltpu.VMEM((SPW // 2, 2 * D), jnp.float32),
            pltpu.SemaphoreType.DMA,
            pltpu.SemaphoreType.DMA,
            pltpu.SemaphoreType.DMA,
            pltpu.SemaphoreType.DMA,
        ],
        compiler_params=pltpu.CompilerParams(use_tc_tiling_on_sc=False),
    )
    def k(data_hbm, emb_hbm, out_hbm, idxc0, idxc1, rows0, rows1, ostage,
          gsem0, gsem1, isem0, isem1):
        cid = lax.axis_index("c")
        sid = lax.axis_index("s")
        wid = sid * 2 + cid
        base = wid * SPW

        idxc = (idxc0, idxc1)
        rows_bufs = (rows0, rows1)
        gsems = (gsem0, gsem1)
        isems = (isem0, isem1)

        def stage_chunk(c, par):
            pltpu.async_copy(
                data_hbm.at[pl.ds(base + c * IDXB, IDXB)], idxc[par],
                isems[par])

        def wait_chunk(c, par):
            pltpu.make_async_copy(
                data_hbm.at[pl.ds(base + c * IDXB, IDXB)], idxc[par],
                isems[par]).wait()

        def fire_seg(ls, b, par):
            for j in range(2):
                o, n = SPLITS[j], SPLITS[j + 1] - SPLITS[j]
                pltpu.async_copy(
                    emb_hbm.at[idxc[par].at[ls, pl.ds(o, n)]],
                    rows_bufs[b].at[pl.ds(o, n)],
                    gsems[b])

        def wait_gathers(ls, b, par):
            for j in range(2):
                o, n = SPLITS[j], SPLITS[j + 1] - SPLITS[j]
                pltpu.make_async_copy(
                    emb_hbm.at[idxc[par].at[ls, pl.ds(o, n)]],
                    rows_bufs[b].at[pl.ds(o, n)],
                    gsems[b]).wait()

        def accumulate(row, col, b):
            rows = rows_bufs[b]

            def body(i, acc):
                accs = list(acc)
                for u in range(UNROLL):
                    r = i * UNROLL + u
                    for q in range(4):
                        accs[q] = accs[q] + rows[r, pl.ds(q * 16, 16)]
                return tuple(accs)

            z = jnp.zeros((16,), jnp.float32)
            a = lax.fori_loop(0, L // UNROLL, body, (z, z, z, z))
            for q in range(4):
                ostage[row, pl.ds(col + q * 16, 16)] = a[q] * (1.0 / L)

        stage_chunk(0, 0)

        @pl.loop(0, NCH, step=2)
        def _(c2):
            for cp in range(2):
                c = c2 + cp
                cbase = c * IDXB

                @pl.when(c + 1 < NCH)
                def _():
                    stage_chunk(c + 1, 1 - cp)

                wait_chunk(c, cp)
                fire_seg(0, 0, cp)

                @pl.loop(0, IDXB // 2)
                def _(p):
                    s0 = p * 2
                    orow = cbase // 2 + p
                    fire_seg(s0 + 1, 1, cp)
                    wait_gathers(s0, 0, cp)
                    accumulate(orow, 0, 0)

                    @pl.when(p < IDXB // 2 - 1)
                    def _():
                        fire_seg(s0 + 2, 0, cp)

                    wait_gathers(s0 + 1, 1, cp)
                    accumulate(orow, D, 1)

        pltpu.sync_copy(ostage, out_hbm.at[pl.ds(wid * (SPW // 2), SPW // 2)])

    return k(data_r, emb)


def _repack_table(emb):
    def body(x_ref, o_ref):
        xt = x_ref[...].T
        o_ref[...] = jnp.concatenate([xt[: RP_CB // 2], xt[RP_CB // 2:]],
                                     axis=1)

    return pl.pallas_call(
        body,
        grid=(VPAD // RP_CB,),
        in_specs=[pl.BlockSpec((D, RP_CB), lambda i: (0, i))],
        out_specs=pl.BlockSpec((RP_CB // 2, 2 * D), lambda i: (i, 0)),
        out_shape=jax.ShapeDtypeStruct((VPAD // 2, 2 * D), jnp.float32),
    )(emb.T)


def _fold_weights(W1, b1r, W2, b2r):
    def body(w1_ref, b1_ref, w2_ref, b2_ref, w12_ref, bias_ref):
        w12_ref[...] = jnp.dot(w1_ref[...], w2_ref[...],
                               preferred_element_type=jnp.float32)
        bias_ref[...] = jnp.dot(b1_ref[...], w2_ref[...],
                                preferred_element_type=jnp.float32) + b2_ref[...]

    return pl.pallas_call(
        body,
        out_shape=(jax.ShapeDtypeStruct((D, H2), jnp.float32),
                   jax.ShapeDtypeStruct((1, H2), jnp.float32)),
    )(W1, b1r, W2, b2r)


def _mlp(x, w12, bias):
    blk = 1024

    def body(x_ref, w12_ref, bias_ref, o_ref):
        o_ref[...] = jnp.dot(x_ref[...], w12_ref[...],
                             preferred_element_type=jnp.float32) + bias_ref[...]

    return pl.pallas_call(
        body,
        grid=(S // blk,),
        in_specs=[
            pl.BlockSpec((blk, D), lambda i: (i, 0)),
            pl.BlockSpec((D, H2), lambda i: (0, 0)),
            pl.BlockSpec((1, H2), lambda i: (0, 0)),
        ],
        out_specs=pl.BlockSpec((blk, H2), lambda i: (i, 0)),
        out_shape=jax.ShapeDtypeStruct((S, H2), jnp.float32),
    )(x, w12, bias)


def kernel(data, emb, W1, b1, W2, b2):
    d = data.astype(jnp.int32)
    d = (d & ~8191) | ((d & 4095) << 1) | ((d >> 12) & 1)
    data_t = jnp.transpose(d, (1, 0, 2))
    data_r = jax.lax.optimization_barrier(data_t.reshape(-1)).reshape(S, L)
    emb_op = _repack_table(emb).reshape(-1).reshape(VPAD, D)
    x = _sc_mean(data_r, emb_op).reshape(S, D)
    w12, bias = _fold_weights(W1, b1.reshape(1, H1), W2, b2.reshape(1, H2))
    out = _mlp(x, w12, bias)
    return out.reshape(NA, B, H2).transpose(1, 0, 2)


# speedup vs baseline: 3.8105x; 1.2933x over previous
import functools

import jax
import jax.numpy as jnp
from jax import lax
from jax.experimental import pallas as pl
from jax.experimental.pallas import tpu as pltpu
from jax.experimental.pallas import tpu_sc as plsc

B, NA, L = 1024, 18, 200
D = 64
H1, H2 = 2048, 512
VOCAB = 1000000
RP_CB = 8192
VPAD = 123 * RP_CB
S = B * NA
NW = 32
SPW = S // NW
SPLITS = (0, 104, 200)
UNROLL = 20
NBUF = 4
IDXB = 48
NCH = SPW // IDXB


def _sc_mean(data_r, emb):
    mesh = plsc.VectorSubcoreMesh(core_axis_name="c", subcore_axis_name="s")

    @functools.partial(
        pl.kernel,
        out_type=jax.ShapeDtypeStruct((S // 2, 2 * D), jnp.float32),
        mesh=mesh,
        scratch_types=[
            pltpu.VMEM((IDXB, L), jnp.int32),
            pltpu.VMEM((IDXB, L), jnp.int32),
            pltpu.VMEM((L, D), jnp.float32),
            pltpu.VMEM((L, D), jnp.float32),
            pltpu.VMEM((L, D), jnp.float32),
            pltpu.VMEM((L, D), jnp.float32),
            pltpu.VMEM((SPW // 2, 2 * D), jnp.float32),
            pltpu.SemaphoreType.DMA,
            pltpu.SemaphoreType.DMA,
            pltpu.SemaphoreType.DMA,
            pltpu.SemaphoreType.DMA,
            pltpu.SemaphoreType.DMA,
            pltpu.SemaphoreType.DMA,
        ],
        compiler_params=pltpu.CompilerParams(use_tc_tiling_on_sc=False),
    )
    def k(data_hbm, emb_hbm, out_hbm, idxc0, idxc1, rows0, rows1, rows2,
          rows3, ostage, gsem0, gsem1, gsem2, gsem3, isem0, isem1):
        cid = lax.axis_index("c")
        sid = lax.axis_index("s")
        wid = sid * 2 + cid
        base = wid * SPW

        idxc = (idxc0, idxc1)
        rows_bufs = (rows0, rows1, rows2, rows3)
        gsems = (gsem0, gsem1, gsem2, gsem3)
        isems = (isem0, isem1)

        def stage_chunk(c, par):
            pltpu.async_copy(
                data_hbm.at[pl.ds(base + c * IDXB, IDXB)], idxc[par],
                isems[par])

        def wait_chunk(c, par):
            pltpu.make_async_copy(
                data_hbm.at[pl.ds(base + c * IDXB, IDXB)], idxc[par],
                isems[par]).wait()

        def fire_seg(ls, b, par):
            for j in range(2):
                o, n = SPLITS[j], SPLITS[j + 1] - SPLITS[j]
                pltpu.async_copy(
                    emb_hbm.at[idxc[par].at[ls, pl.ds(o, n)]],
                    rows_bufs[b].at[pl.ds(o, n)],
                    gsems[b])

        def wait_gathers(ls, b, par):
            for j in range(2):
                o, n = SPLITS[j], SPLITS[j + 1] - SPLITS[j]
                pltpu.make_async_copy(
                    emb_hbm.at[idxc[par].at[ls, pl.ds(o, n)]],
                    rows_bufs[b].at[pl.ds(o, n)],
                    gsems[b]).wait()

        def accumulate(row, col, b):
            rows = rows_bufs[b]

            def body(i, acc):
                accs = list(acc)
                for u in range(UNROLL):
                    r = i * UNROLL + u
                    for q in range(4):
                        a = (u % 2) * 4 + q
                        accs[a] = accs[a] + rows[r, pl.ds(q * 16, 16)]
                return tuple(accs)

            z = jnp.zeros((16,), jnp.float32)
            a = lax.fori_loop(0, L // UNROLL, body, (z,) * 8)
            for q in range(4):
                ostage[row, pl.ds(col + q * 16, 16)] = (
                    (a[q] + a[q + 4]) * (1.0 / L))

        stage_chunk(0, 0)
        wait_chunk(0, 0)
        for k in range(NBUF):
            fire_seg(k, k, 0)

        NQ = IDXB // NBUF

        @pl.loop(0, NCH, step=2)
        def _(c2):
            for cp in range(2):
                c = c2 + cp
                cbase = c * IDXB
                nxt = 1 - cp
                last = (c2 >= NCH - 2) if cp else False

                def _pref():
                    stage_chunk(c + 1, nxt)
                if cp == 0:
                    _pref()
                else:
                    pl.when(jnp.logical_not(last))(_pref)

                @pl.loop(0, NQ - 1)
                def _(q):
                    s0 = NBUF * q
                    for k in range(NBUF):
                        s = s0 + k
                        wait_gathers(s, k, cp)
                        accumulate(cbase // 2 + (s0 + k) // 2, (k % 2) * D, k)
                        fire_seg(s + NBUF, k, cp)

                def _wc():
                    wait_chunk(c + 1, nxt)
                if cp == 0:
                    _wc()
                else:
                    pl.when(jnp.logical_not(last))(_wc)

                s0 = IDXB - NBUF
                for k in range(NBUF):
                    wait_gathers(s0 + k, k, cp)
                    accumulate(cbase // 2 + (s0 + k) // 2, (k % 2) * D, k)

                    def _f(k=k):
                        fire_seg(k, k, nxt)
                    if cp == 0:
                        _f()
                    else:
                        pl.when(jnp.logical_not(last))(_f)

        pltpu.sync_copy(ostage, out_hbm.at[pl.ds(wid * (SPW // 2), SPW // 2)])

    return k(data_r, emb)


def _repack_table(emb):
    def body(x_ref, o_ref):
        xt = x_ref[...].T
        o_ref[...] = jnp.concatenate([xt[: RP_CB // 2], xt[RP_CB // 2:]],
                                     axis=1)

    return pl.pallas_call(
        body,
        grid=(VPAD // RP_CB,),
        in_specs=[pl.BlockSpec((D, RP_CB), lambda i: (0, i))],
        out_specs=pl.BlockSpec((RP_CB // 2, 2 * D), lambda i: (i, 0)),
        out_shape=jax.ShapeDtypeStruct((VPAD // 2, 2 * D), jnp.float32),
    )(emb.T)


def _fold_weights(W1, b1r, W2, b2r):
    def body(w1_ref, b1_ref, w2_ref, b2_ref, w12_ref, bias_ref):
        w12_ref[...] = jnp.dot(w1_ref[...], w2_ref[...],
                               preferred_element_type=jnp.float32)
        bias_ref[...] = jnp.dot(b1_ref[...], w2_ref[...],
                                preferred_element_type=jnp.float32) + b2_ref[...]

    return pl.pallas_call(
        body,
        out_shape=(jax.ShapeDtypeStruct((D, H2), jnp.float32),
                   jax.ShapeDtypeStruct((1, H2), jnp.float32)),
    )(W1, b1r, W2, b2r)


def _mlp(x, w12, bias):
    blk = 1024

    def body(x_ref, w12_ref, bias_ref, o_ref):
        o_ref[...] = jnp.dot(x_ref[...], w12_ref[...],
                             preferred_element_type=jnp.float32) + bias_ref[...]

    return pl.pallas_call(
        body,
        grid=(S // blk,),
        in_specs=[
            pl.BlockSpec((blk, D), lambda i: (i, 0)),
            pl.BlockSpec((D, H2), lambda i: (0, 0)),
            pl.BlockSpec((1, H2), lambda i: (0, 0)),
        ],
        out_specs=pl.BlockSpec((blk, H2), lambda i: (i, 0)),
        out_shape=jax.ShapeDtypeStruct((S, H2), jnp.float32),
    )(x, w12, bias)


def kernel(data, emb, W1, b1, W2, b2):
    d = data.astype(jnp.int32)
    d = (d & ~8191) | ((d & 4095) << 1) | ((d >> 12) & 1)
    data_t = jnp.transpose(d, (1, 0, 2))
    data_r = jax.lax.optimization_barrier(data_t.reshape(-1)).reshape(S, L)
    emb_op = _repack_table(emb).reshape(-1).reshape(VPAD, D)
    x = _sc_mean(data_r, emb_op).reshape(S, D)
    w12, bias = _fold_weights(W1, b1.reshape(1, H1), W2, b2.reshape(1, H2))
    out = _mlp(x, w12, bias)
    return out.reshape(NA, B, H2).transpose(1, 0, 2)


# speedup vs baseline: 4.5421x; 1.1920x over previous
import functools

import jax
import jax.numpy as jnp
from jax import lax
from jax.experimental import pallas as pl
from jax.experimental.pallas import tpu as pltpu
from jax.experimental.pallas import tpu_sc as plsc

B, NA, L = 1024, 18, 200
D = 64
H1, H2 = 2048, 512
VOCAB = 1000000
RP_CB = 8192
VPAD = 123 * RP_CB
S = B * NA
NW = 32
SPW = S // NW
SPLITS = (0, 104, 200)
UNROLL = 20
NBUF = 4
IDXB = 48
NCH = SPW // IDXB


def _sc_mean(data_r, emb):
    mesh = plsc.VectorSubcoreMesh(core_axis_name="c", subcore_axis_name="s")

    @functools.partial(
        pl.kernel,
        out_type=jax.ShapeDtypeStruct((S // 2, 2 * D), jnp.float32),
        mesh=mesh,
        scratch_types=[
            pltpu.VMEM((IDXB, L), jnp.int32),
            pltpu.VMEM((IDXB, L), jnp.int32),
            pltpu.VMEM((L, D // 2), jnp.float32),
            pltpu.VMEM((L, D // 2), jnp.float32),
            pltpu.VMEM((L, D // 2), jnp.float32),
            pltpu.VMEM((L, D // 2), jnp.float32),
            pltpu.VMEM((SPW // 2, 2 * D), jnp.float32),
            pltpu.SemaphoreType.DMA,
            pltpu.SemaphoreType.DMA,
            pltpu.SemaphoreType.DMA,
            pltpu.SemaphoreType.DMA,
            pltpu.SemaphoreType.DMA,
            pltpu.SemaphoreType.DMA,
        ],
        compiler_params=pltpu.CompilerParams(use_tc_tiling_on_sc=False,
                                             needs_layout_passes=False),
    )
    def k(data_hbm, emb_hbm, out_hbm, idxc0, idxc1, rows0, rows1, rows2,
          rows3, ostage, gsem0, gsem1, gsem2, gsem3, isem0, isem1):
        cid = lax.axis_index("c")
        sid = lax.axis_index("s")
        wid = sid * 2 + cid
        base = wid * SPW

        idxc = (idxc0, idxc1)
        rows_bufs = (rows0, rows1, rows2, rows3)
        gsems = (gsem0, gsem1, gsem2, gsem3)
        isems = (isem0, isem1)

        def stage_chunk(c, par):
            pltpu.async_copy(
                data_hbm.at[pl.ds(base + c * IDXB, IDXB)], idxc[par],
                isems[par])

        def wait_chunk(c, par):
            pltpu.make_async_copy(
                data_hbm.at[pl.ds(base + c * IDXB, IDXB)], idxc[par],
                isems[par]).wait()

        def fire_seg(ls, b, par):
            for j in range(2):
                o, n = SPLITS[j], SPLITS[j + 1] - SPLITS[j]
                pltpu.async_copy(
                    emb_hbm.at[idxc[par].at[ls, pl.ds(o, n)]],
                    rows_bufs[b].at[pl.ds(o, n)],
                    gsems[b])

        def wait_gathers(ls, b, par):
            for j in range(2):
                o, n = SPLITS[j], SPLITS[j + 1] - SPLITS[j]
                pltpu.make_async_copy(
                    emb_hbm.at[idxc[par].at[ls, pl.ds(o, n)]],
                    rows_bufs[b].at[pl.ds(o, n)],
                    gsems[b]).wait()

        def accumulate(row, col, b):
            rows = rows_bufs[b]

            def body(i, acc):
                accs = list(acc)
                for u in range(UNROLL):
                    r = i * UNROLL + u
                    for q in range(2):
                        w = rows[r, pl.ds(q * 16, 16)]
                        lo, hi = plsc.unpack(
                            plsc.bitcast(w, jnp.bfloat16),
                            format=plsc.PackFormat.INTERLEAVED)
                        a = (u % 2) * 4
                        accs[a + q] = accs[a + q] + lo
                        accs[a + 2 + q] = accs[a + 2 + q] + hi
                return tuple(accs)

            z = jnp.zeros((16,), jnp.float32)
            a = lax.fori_loop(0, L // UNROLL, body, (z,) * 8)
            for q in range(4):
                ostage[row, pl.ds(col + q * 16, 16)] = (
                    (a[q] + a[q + 4]) * (1.0 / L))

        stage_chunk(0, 0)
        wait_chunk(0, 0)
        for k in range(NBUF):
            fire_seg(k, k, 0)

        NQ = IDXB // NBUF

        @pl.loop(0, NCH, step=2)
        def _(c2):
            for cp in range(2):
                c = c2 + cp
                cbase = c * IDXB
                nxt = 1 - cp
                last = (c2 >= NCH - 2) if cp else False

                def _pref():
                    stage_chunk(c + 1, nxt)
                if cp == 0:
                    _pref()
                else:
                    pl.when(jnp.logical_not(last))(_pref)

                @pl.loop(0, NQ - 1)
                def _(q):
                    s0 = NBUF * q
                    for k in range(NBUF):
                        s = s0 + k
                        wait_gathers(s, k, cp)
                        accumulate(cbase // 2 + (s0 + k) // 2, (k % 2) * D, k)
                        fire_seg(s + NBUF, k, cp)

                def _wc():
                    wait_chunk(c + 1, nxt)
                if cp == 0:
                    _wc()
                else:
                    pl.when(jnp.logical_not(last))(_wc)

                s0 = IDXB - NBUF
                for k in range(NBUF):
                    wait_gathers(s0 + k, k, cp)
                    accumulate(cbase // 2 + (s0 + k) // 2, (k % 2) * D, k)

                    def _f(k=k):
                        fire_seg(k, k, nxt)
                    if cp == 0:
                        _f()
                    else:
                        pl.when(jnp.logical_not(last))(_f)

        pltpu.sync_copy(ostage, out_hbm.at[pl.ds(wid * (SPW // 2), SPW // 2)])

    return k(data_r, emb)


def _repack_table(emb):
    q = RP_CB // 4

    def body(x_ref, o_ref):
        xt = x_ref[...].T
        lo = jax.lax.bitcast_convert_type(
            xt[:, :D // 2].astype(jnp.bfloat16), jnp.uint16).astype(jnp.uint32)
        hi = jax.lax.bitcast_convert_type(
            xt[:, D // 2:].astype(jnp.bfloat16), jnp.uint16).astype(jnp.uint32)
        w = jax.lax.bitcast_convert_type(lo | (hi << 16), jnp.float32)
        o_ref[...] = jnp.concatenate(
            [w[k * q:(k + 1) * q] for k in range(4)], axis=1)

    return pl.pallas_call(
        body,
        grid=(VPAD // RP_CB,),
        in_specs=[pl.BlockSpec((D, RP_CB), lambda i: (0, i))],
        out_specs=pl.BlockSpec((q, 2 * D), lambda i: (i, 0)),
        out_shape=jax.ShapeDtypeStruct((VPAD // 4, 2 * D), jnp.float32),
    )(emb.T)


def _fold_weights(W1, b1r, W2, b2r):
    def body(w1_ref, b1_ref, w2_ref, b2_ref, w12_ref, bias_ref):
        w12_ref[...] = jnp.dot(w1_ref[...], w2_ref[...],
                               preferred_element_type=jnp.float32)
        bias_ref[...] = jnp.dot(b1_ref[...], w2_ref[...],
                                preferred_element_type=jnp.float32) + b2_ref[...]

    return pl.pallas_call(
        body,
        out_shape=(jax.ShapeDtypeStruct((D, H2), jnp.float32),
                   jax.ShapeDtypeStruct((1, H2), jnp.float32)),
    )(W1, b1r, W2, b2r)


def _mlp(x, w12, bias):
    blk = 1024

    def body(x_ref, w12_ref, bias_ref, o_ref):
        o_ref[...] = jnp.dot(x_ref[...], w12_ref[...],
                             preferred_element_type=jnp.float32) + bias_ref[...]

    return pl.pallas_call(
        body,
        grid=(S // blk,),
        in_specs=[
            pl.BlockSpec((blk, D), lambda i: (i, 0)),
            pl.BlockSpec((D, H2), lambda i: (0, 0)),
            pl.BlockSpec((1, H2), lambda i: (0, 0)),
        ],
        out_specs=pl.BlockSpec((blk, H2), lambda i: (i, 0)),
        out_shape=jax.ShapeDtypeStruct((S, H2), jnp.float32),
    )(x, w12, bias)


def kernel(data, emb, W1, b1, W2, b2):
    d = data.astype(jnp.int32)
    d = (d & ~8191) | ((d & 2047) << 2) | ((d >> 11) & 3)
    data_t = jnp.transpose(d, (1, 0, 2))
    data_r = jax.lax.optimization_barrier(data_t.reshape(-1)).reshape(S, L)
    emb_op = _repack_table(emb).reshape(-1).reshape(VPAD, D // 2)
    x = _sc_mean(data_r, emb_op).reshape(S, D)
    w12, bias = _fold_weights(W1, b1.reshape(1, H1), W2, b2.reshape(1, H2))
    out = _mlp(x, w12, bias)
    return out.reshape(NA, B, H2).transpose(1, 0, 2)
